# Initial kernel scaffold; baseline (speedup 1.0000x reference)
import jax
import jax.numpy as jnp
from jax.experimental import pallas as pl


def kernel(input_ids, tree_ids, emb, W_ioux, W_iouh0, b_iouh0, W_iouh1, b_iouh1, W_fx, W_fh0, b_fh0, W_fh1, b_fh1, W_fh2, b_fh2, W_fh3, b_fh3):
    raise NotImplementedError("write your pallas kernel here")


# speedup vs baseline: 1.0004x; 1.0004x over previous
import jax
import jax.numpy as jnp
from jax.experimental import pallas as pl

B, S, H, E, NSTEPS = 1024, 50, 64, 64, 10


def _emb_matmul_kernel(x_ref, wioux_ref, wfx_ref, iou_ref, fx_ref):
    x = x_ref[...]
    iou_ref[...] = jax.lax.dot_general(
        x, wioux_ref[...], (((1,), (1,)), ((), ())),
        preferred_element_type=jnp.float32)
    fx_ref[...] = jax.lax.dot_general(
        x, wfx_ref[...], (((1,), (1,)), ((), ())),
        preferred_element_type=jnp.float32)


def kernel(input_ids, tree_ids, emb, W_ioux, W_iouh0, b_iouh0, W_iouh1, b_iouh1,
           W_fx, W_fh0, b_fh0, W_fh1, b_fh1, W_fh2, b_fh2, W_fh3, b_fh3):
    x = jnp.take(emb, input_ids, axis=0)
    xf = x.reshape(B * S, E)

    NBLK = 64
    RB = (B * S) // NBLK
    iou_x_f, fx_f = pl.pallas_call(
        _emb_matmul_kernel,
        grid=(NBLK,),
        in_specs=[
            pl.BlockSpec((RB, E), lambda i: (i, 0)),
            pl.BlockSpec((3 * H, E), lambda i: (0, 0)),
            pl.BlockSpec((H, E), lambda i: (0, 0)),
        ],
        out_specs=[
            pl.BlockSpec((RB, 3 * H), lambda i: (i, 0)),
            pl.BlockSpec((RB, H), lambda i: (i, 0)),
        ],
        out_shape=[
            jax.ShapeDtypeStruct((B * S, 3 * H), jnp.float32),
            jax.ShapeDtypeStruct((B * S, H), jnp.float32),
        ],
    )(xf, W_ioux, W_fx)
    iou_x = iou_x_f.reshape(B, S, 3 * H)
    fx = fx_f.reshape(B, S, H)

    h = jnp.zeros((B, S, H), dtype=x.dtype)
    c = jnp.zeros((B, S, H), dtype=x.dtype)
    b_idx = jnp.arange(B)[:, None, None]
    h_idx = jnp.arange(H)[None, None, :]

    def expand(t):
        return jnp.broadcast_to(t[:, :, None], (B, S, H))

    def gather1(src, idx):
        return src[b_idx, idx, h_idx]

    for step in range(NSTEPS):
        idx_d = expand(tree_ids[step, 0])
        idx_r = expand(tree_ids[step, 1])
        idx_l = expand(tree_ids[step, 2])
        iou_hr = h @ W_iouh0.T + b_iouh0
        iou_hl = h @ W_iouh1.T + b_iouh1
        z = jnp.zeros((B, S, 3 * H), dtype=x.dtype)
        sc_r = z.at[b_idx, idx_r, h_idx].add(iou_hr[..., :H])
        sc_l = z.at[b_idx, idx_l, h_idx].add(iou_hl[..., :H])
        iou = iou_x + sc_r + sc_l
        i, o, u = jnp.split(iou, 3, axis=-1)
        i = jax.nn.sigmoid(i)
        o = jax.nn.sigmoid(o)
        u = jnp.tanh(u)
        f = gather1(fx, idx_d)
        f = f + gather1(h @ W_fh0.T + b_fh0, idx_r)
        f = f + gather1(h @ W_fh1.T + b_fh1, idx_r)
        f = f + gather1(h @ W_fh2.T + b_fh2, idx_l)
        f = f + gather1(h @ W_fh3.T + b_fh3, idx_l)
        f = jax.nn.sigmoid(f)
        fc = f * c
        c_new = i * u + jnp.zeros_like(fc).at[b_idx, idx_d, h_idx].add(fc)
        h_new = o * jnp.tanh(c_new)
        mask = (idx_d != 0).reshape(-1)
        src_pos = jnp.cumsum(mask) - 1
        h = jnp.where(mask, h_new.reshape(-1)[src_pos], h.reshape(-1)).reshape(B, S, H)
        c = jnp.where(mask, c_new.reshape(-1)[src_pos], c.reshape(-1)).reshape(B, S, H)
    return h


# speedup vs baseline: 42.4499x; 42.4326x over previous
import functools

import jax
import jax.numpy as jnp
from jax import lax
from jax.experimental import pallas as pl
from jax.experimental.pallas import tpu as pltpu
from jax.experimental.pallas import tpu_sc as plsc

B, S, H, E, V, NSTEPS = 1024, 50, 64, 64, 100000, 10
BS = B * S
NW = 32
NPER = BS // NW
R = 200
NBLK = BS // R

_F32 = jnp.float32


def _sc_chunks():
    out, off = [], 0
    while off < NPER:
        sz = min(128, NPER - off)
        out.append((off, sz))
        off += sz
    return out


def _sc_gather_one_body(tab_hbm, idx_hbm, out_hbm, idx_v, rows_v, sem):
    wid = lax.axis_index("s") * 2 + lax.axis_index("c")
    base = wid * NPER
    pltpu.sync_copy(idx_hbm.at[pl.ds(base, NPER)], idx_v)
    cps = [
        pltpu.async_copy(tab_hbm.at[idx_v.at[pl.ds(off, sz)]],
                         rows_v.at[pl.ds(off, sz)], sem)
        for off, sz in _sc_chunks()
    ]
    for cp in cps:
        cp.wait()
    pltpu.sync_copy(rows_v, out_hbm.at[pl.ds(base, NPER)])


def _sc_gather_pair_body(tabh_hbm, tabc_hbm, idx_hbm, outh_hbm, outc_hbm,
                         idx_v, rows_v, sem):
    wid = lax.axis_index("s") * 2 + lax.axis_index("c")
    base = wid * NPER
    pltpu.sync_copy(idx_hbm.at[pl.ds(base, NPER)], idx_v)
    for tab, out in ((tabh_hbm, outh_hbm), (tabc_hbm, outc_hbm)):
        cps = [
            pltpu.async_copy(tab.at[idx_v.at[pl.ds(off, sz)]],
                             rows_v.at[pl.ds(off, sz)], sem)
            for off, sz in _sc_chunks()
        ]
        for cp in cps:
            cp.wait()
        pltpu.sync_copy(rows_v, out.at[pl.ds(base, NPER)])


def _sc_mesh():
    return plsc.VectorSubcoreMesh(core_axis_name="c", subcore_axis_name="s")


_SC_PARAMS = pltpu.CompilerParams(use_tc_tiling_on_sc=False)


def _sc_gather_rows(table, idx):
    return pl.kernel(
        _sc_gather_one_body,
        out_type=jax.ShapeDtypeStruct((BS, H), _F32),
        mesh=_sc_mesh(),
        compiler_params=_SC_PARAMS,
        scratch_types=[
            pltpu.VMEM((NPER,), jnp.int32),
            pltpu.VMEM((NPER, H), _F32),
            pltpu.SemaphoreType.DMA,
        ],
    )(table, idx)


def _sc_gather_rows_pair(tab_h, tab_c, idx):
    return pl.kernel(
        _sc_gather_pair_body,
        out_type=[jax.ShapeDtypeStruct((BS, H), _F32),
                  jax.ShapeDtypeStruct((BS, H), _F32)],
        mesh=_sc_mesh(),
        compiler_params=_SC_PARAMS,
        scratch_types=[
            pltpu.VMEM((NPER,), jnp.int32),
            pltpu.VMEM((NPER, H), _F32),
            pltpu.SemaphoreType.DMA,
        ],
    )(tab_h, tab_c, idx)


def _pre_body(x_ref, wpre_ref, pre_ref):
    t = jax.lax.dot_general(x_ref[...], wpre_ref[...],
                            (((1,), (0,)), ((), ())),
                            preferred_element_type=_F32)
    col = lax.broadcasted_iota(jnp.int32, t.shape, 1)
    pre_ref[...] = jnp.where(
        col < 64, t,
        jnp.where(col < 128, jax.nn.sigmoid(t),
                  jnp.where(col < 192, jnp.tanh(t), t)))


def _tc_preamble(x, wpre_t):
    rp = 800
    return pl.pallas_call(
        _pre_body,
        grid=(BS // rp,),
        in_specs=[pl.BlockSpec((rp, E), lambda k: (k, 0)),
                  pl.BlockSpec((E, 4 * H), lambda k: (0, 0))],
        out_specs=pl.BlockSpec((rp, 4 * H), lambda k: (k, 0)),
        out_shape=jax.ShapeDtypeStruct((BS, 4 * H), _F32),
    )(x, wpre_t)


def _step_body(h_ref, c_ref, pre_ref, tid_ref, w4_ref, b4_ref,
               tabh_ref, tabc_ref, src_ref, carry_ref):
    k = pl.program_id(0)
    h_blk = h_ref[...]
    c_blk = c_ref[...]
    pre = pre_ref[...]

    hh = jax.lax.dot_general(h_blk, w4_ref[...], (((1,), (0,)), ((), ())),
                             preferred_element_type=_F32) + b4_ref[...]
    hr, hl = hh[:, 0:H], hh[:, H:2 * H]
    fr, fl = hh[:, 2 * H:3 * H], hh[:, 3 * H:4 * H]

    ri = lax.broadcasted_iota(jnp.int32, (R, 1), 0)
    ebase = (ri // S) * S
    cj = lax.broadcasted_iota(jnp.int32, (R, R), 1)

    def onehot(idx_col):
        return (jnp.broadcast_to(ebase + idx_col, (R, R)) == cj).astype(_F32)

    idx_d = tid_ref[:, 0:1]
    o_d = onehot(idx_d)
    o_r = onehot(tid_ref[:, 1:2])
    o_l = onehot(tid_ref[:, 2:3])

    def gath(o, v):
        return jax.lax.dot_general(o, v, (((1,), (0,)), ((), ())),
                                   preferred_element_type=_F32)

    def scat(o, v):
        return jax.lax.dot_general(o, v, (((0,), (0,)), ((), ())),
                                   preferred_element_type=_F32)

    i_gate = jax.nn.sigmoid(pre[:, 0:H] + scat(o_r, hr) + scat(o_l, hl))
    o_gate = pre[:, H:2 * H]
    u_gate = pre[:, 2 * H:3 * H]
    fx_blk = pre[:, 3 * H:4 * H]
    f_gate = jax.nn.sigmoid(gath(o_d, fx_blk) + gath(o_r, fr) + gath(o_l, fl))
    c_new = i_gate * u_gate + scat(o_d, f_gate * c_blk)
    h_new = o_gate * jnp.tanh(c_new)

    tabh_ref[0] = h_new
    tabh_ref[1] = h_blk
    tabc_ref[0] = c_new
    tabc_ref[1] = c_blk

    mask = idx_d != 0
    mask_f = mask.astype(_F32)
    tri = (cj <= lax.broadcasted_iota(jnp.int32, (R, R), 0)).astype(_F32)
    cum = jax.lax.dot_general(tri, mask_f, (((1,), (0,)), ((), ())),
                              preferred_element_type=_F32).astype(jnp.int32)
    base = jnp.where(k == 0, 0, carry_ref[0])
    carry_ref[0] = base + jnp.sum(mask_f).astype(jnp.int32)
    rank = base + cum - 1
    src_ref[...] = jnp.where(mask, rank, BS + k * R + ri)


def _tc_step(h, c, pre, tid, w4_t, b4):
    return pl.pallas_call(
        _step_body,
        grid=(NBLK,),
        in_specs=[
            pl.BlockSpec((R, H), lambda k: (k, 0)),
            pl.BlockSpec((R, H), lambda k: (k, 0)),
            pl.BlockSpec((R, 4 * H), lambda k: (k, 0)),
            pl.BlockSpec((R, 3), lambda k: (k, 0)),
            pl.BlockSpec((H, 4 * H), lambda k: (0, 0)),
            pl.BlockSpec((1, 4 * H), lambda k: (0, 0)),
        ],
        out_specs=[
            pl.BlockSpec((2, R, H), lambda k: (0, k, 0)),
            pl.BlockSpec((2, R, H), lambda k: (0, k, 0)),
            pl.BlockSpec((R, 1), lambda k: (k, 0)),
        ],
        out_shape=[
            jax.ShapeDtypeStruct((2, BS, H), _F32),
            jax.ShapeDtypeStruct((2, BS, H), _F32),
            jax.ShapeDtypeStruct((BS, 1), jnp.int32),
        ],
        scratch_shapes=[pltpu.SMEM((1,), jnp.int32)],
    )(h, c, pre, tid, w4_t, b4)


def kernel(input_ids, tree_ids, emb, W_ioux, W_iouh0, b_iouh0, W_iouh1,
           b_iouh1, W_fx, W_fh0, b_fh0, W_fh1, b_fh1, W_fh2, b_fh2,
           W_fh3, b_fh3):
    ids = input_ids.reshape(BS).astype(jnp.int32)
    x = _sc_gather_rows(emb, ids)

    wpre_t = jnp.concatenate([W_ioux, W_fx], axis=0).T
    pre = _tc_preamble(x, wpre_t)

    w4 = jnp.concatenate([W_iouh0[:H], W_iouh1[:H],
                          W_fh0 + W_fh1, W_fh2 + W_fh3], axis=0)
    b4 = jnp.concatenate([b_iouh0[:H], b_iouh1[:H],
                          b_fh0 + b_fh1, b_fh2 + b_fh3]).reshape(1, 4 * H)
    w4_t = w4.T

    tids = jnp.transpose(tree_ids.reshape(NSTEPS, 3, BS), (0, 2, 1))
    tids = tids.astype(jnp.int32)

    h = jnp.zeros((BS, H), _F32)
    c = jnp.zeros((BS, H), _F32)
    for step in range(NSTEPS):
        tab_h, tab_c, src = _tc_step(h, c, pre, tids[step], w4_t, b4)
        h, c = _sc_gather_rows_pair(tab_h.reshape(2 * BS, H),
                                    tab_c.reshape(2 * BS, H),
                                    src.reshape(BS))
    return h.reshape(B, S, H)


# speedup vs baseline: 42.7271x; 1.0065x over previous
import functools

import jax
import jax.numpy as jnp
from jax import lax
from jax.experimental import pallas as pl
from jax.experimental.pallas import tpu as pltpu
from jax.experimental.pallas import tpu_sc as plsc

B, S, H, E, V, NSTEPS = 1024, 50, 64, 64, 100000, 10
BS = B * S
NW = 32
NPER = BS // NW
R = 200
NBLK = BS // R

_F32 = jnp.float32


def _sc_chunks():
    out, off = [], 0
    while off < NPER:
        sz = min(128, NPER - off)
        out.append((off, sz))
        off += sz
    return out


def _sc_gather_pre_body(tab_hbm, idx_hbm, out_hbm, idx_v, bufs, sem):
    wid = lax.axis_index("s") * 2 + lax.axis_index("c")
    base = wid * NPER
    pltpu.sync_copy(idx_hbm.at[pl.ds(base, NPER)], idx_v)
    chunks = _sc_chunks()
    prev = None
    for t, (off, sz) in enumerate(chunks):
        cur = pltpu.async_copy(tab_hbm.at[idx_v.at[pl.ds(off, sz)]],
                               bufs.at[t % 2, pl.ds(0, sz)], sem)
        if prev is not None:
            poff, psz = chunks[t - 1]
            prev.wait()
            pltpu.sync_copy(bufs.at[(t - 1) % 2, pl.ds(0, psz)],
                            out_hbm.at[pl.ds(base + poff, psz)])
        prev = cur
    loff, lsz = chunks[-1]
    prev.wait()
    pltpu.sync_copy(bufs.at[(len(chunks) - 1) % 2, pl.ds(0, lsz)],
                    out_hbm.at[pl.ds(base + loff, lsz)])


def _sc_gather_pair_body(tabh_hbm, tabc_hbm, idx_hbm, outh_hbm, outc_hbm,
                         idx_v, rows_v, sem):
    wid = lax.axis_index("s") * 2 + lax.axis_index("c")
    base = wid * NPER
    pltpu.sync_copy(idx_hbm.at[pl.ds(base, NPER)], idx_v)
    for tab, out in ((tabh_hbm, outh_hbm), (tabc_hbm, outc_hbm)):
        cps = [
            pltpu.async_copy(tab.at[idx_v.at[pl.ds(off, sz)]],
                             rows_v.at[pl.ds(off, sz)], sem)
            for off, sz in _sc_chunks()
        ]
        for cp in cps:
            cp.wait()
        pltpu.sync_copy(rows_v, out.at[pl.ds(base, NPER)])


def _sc_mesh():
    return plsc.VectorSubcoreMesh(core_axis_name="c", subcore_axis_name="s")


_SC_PARAMS = pltpu.CompilerParams(use_tc_tiling_on_sc=False)


def _sc_gather_pre(table, idx):
    return pl.kernel(
        _sc_gather_pre_body,
        out_type=jax.ShapeDtypeStruct((BS, 4 * H), _F32),
        mesh=_sc_mesh(),
        scratch_types=[
            pltpu.VMEM((NPER,), jnp.int32),
            pltpu.VMEM((2, 128, 4 * H), _F32),
            pltpu.SemaphoreType.DMA,
        ],
    )(table, idx)


def _sc_gather_rows_pair(tab_h, tab_c, idx):
    return pl.kernel(
        _sc_gather_pair_body,
        out_type=[jax.ShapeDtypeStruct((BS, H), _F32),
                  jax.ShapeDtypeStruct((BS, H), _F32)],
        mesh=_sc_mesh(),
        compiler_params=_SC_PARAMS,
        scratch_types=[
            pltpu.VMEM((NPER,), jnp.int32),
            pltpu.VMEM((NPER, H), _F32),
            pltpu.SemaphoreType.DMA,
        ],
    )(tab_h, tab_c, idx)


def _pre_body(x_ref, wpre_ref, pre_ref):
    t = jax.lax.dot_general(x_ref[...], wpre_ref[...],
                            (((1,), (0,)), ((), ())),
                            preferred_element_type=_F32)
    col = lax.broadcasted_iota(jnp.int32, t.shape, 1)
    pre_ref[...] = jnp.where(
        col < 64, t,
        jnp.where(col < 128, jax.nn.sigmoid(t),
                  jnp.where(col < 192, jnp.tanh(t), t)))


def _tc_pre_table(emb, wpre_t):
    rp = 800
    return pl.pallas_call(
        _pre_body,
        grid=(V // rp,),
        in_specs=[pl.BlockSpec((rp, E), lambda k: (k, 0)),
                  pl.BlockSpec((E, 4 * H), lambda k: (0, 0))],
        out_specs=pl.BlockSpec((rp, 4 * H), lambda k: (k, 0)),
        out_shape=jax.ShapeDtypeStruct((V, 4 * H), _F32),
    )(emb, wpre_t)


def _step_body(h_ref, c_ref, pre_ref, tid_ref, w4_ref, b4_ref,
               tabh_ref, tabc_ref, src_ref, carry_ref):
    k = pl.program_id(0)
    h_blk = h_ref[...]
    c_blk = c_ref[...]
    pre = pre_ref[...]

    hh = jax.lax.dot_general(h_blk, w4_ref[...], (((1,), (0,)), ((), ())),
                             preferred_element_type=_F32) + b4_ref[...]
    hr, hl = hh[:, 0:H], hh[:, H:2 * H]
    fr, fl = hh[:, 2 * H:3 * H], hh[:, 3 * H:4 * H]

    ri = lax.broadcasted_iota(jnp.int32, (R, 1), 0)
    ebase = (ri // S) * S
    cj = lax.broadcasted_iota(jnp.int32, (R, R), 1)

    def onehot(idx_col):
        return (jnp.broadcast_to(ebase + idx_col, (R, R)) == cj).astype(_F32)

    idx_d = tid_ref[:, 0:1]
    o_d = onehot(idx_d)
    o_r = onehot(tid_ref[:, 1:2])
    o_l = onehot(tid_ref[:, 2:3])

    def gath(o, v):
        return jax.lax.dot_general(o, v, (((1,), (0,)), ((), ())),
                                   preferred_element_type=_F32)

    def scat(o, v):
        return jax.lax.dot_general(o, v, (((0,), (0,)), ((), ())),
                                   preferred_element_type=_F32)

    i_gate = jax.nn.sigmoid(pre[:, 0:H] + scat(o_r, hr) + scat(o_l, hl))
    o_gate = pre[:, H:2 * H]
    u_gate = pre[:, 2 * H:3 * H]
    fx_blk = pre[:, 3 * H:4 * H]
    f_gate = jax.nn.sigmoid(gath(o_d, fx_blk) + gath(o_r, fr) + gath(o_l, fl))
    c_new = i_gate * u_gate + scat(o_d, f_gate * c_blk)
    h_new = o_gate * jnp.tanh(c_new)

    tabh_ref[0] = h_new
    tabh_ref[1] = h_blk
    tabc_ref[0] = c_new
    tabc_ref[1] = c_blk

    mask = idx_d != 0
    mask_f = mask.astype(_F32)
    tri = (cj <= lax.broadcasted_iota(jnp.int32, (R, R), 0)).astype(_F32)
    cum = jax.lax.dot_general(tri, mask_f, (((1,), (0,)), ((), ())),
                              preferred_element_type=_F32).astype(jnp.int32)
    base = jnp.where(k == 0, 0, carry_ref[0])
    carry_ref[0] = base + jnp.sum(mask_f).astype(jnp.int32)
    rank = base + cum - 1
    src_ref[...] = jnp.where(mask, rank, BS + k * R + ri)


def _tc_step(h, c, pre, tid, w4_t, b4):
    return pl.pallas_call(
        _step_body,
        grid=(NBLK,),
        in_specs=[
            pl.BlockSpec((R, H), lambda k: (k, 0)),
            pl.BlockSpec((R, H), lambda k: (k, 0)),
            pl.BlockSpec((R, 4 * H), lambda k: (k, 0)),
            pl.BlockSpec((R, 3), lambda k: (k, 0)),
            pl.BlockSpec((H, 4 * H), lambda k: (0, 0)),
            pl.BlockSpec((1, 4 * H), lambda k: (0, 0)),
        ],
        out_specs=[
            pl.BlockSpec((2, R, H), lambda k: (0, k, 0)),
            pl.BlockSpec((2, R, H), lambda k: (0, k, 0)),
            pl.BlockSpec((R, 1), lambda k: (k, 0)),
        ],
        out_shape=[
            jax.ShapeDtypeStruct((2, BS, H), _F32),
            jax.ShapeDtypeStruct((2, BS, H), _F32),
            jax.ShapeDtypeStruct((BS, 1), jnp.int32),
        ],
        scratch_shapes=[pltpu.SMEM((1,), jnp.int32)],
    )(h, c, pre, tid, w4_t, b4)


def kernel(input_ids, tree_ids, emb, W_ioux, W_iouh0, b_iouh0, W_iouh1,
           b_iouh1, W_fx, W_fh0, b_fh0, W_fh1, b_fh1, W_fh2, b_fh2,
           W_fh3, b_fh3):
    ids = input_ids.reshape(BS).astype(jnp.int32)
    wpre_t = jnp.concatenate([W_ioux, W_fx], axis=0).T
    pre_table = _tc_pre_table(emb, wpre_t)
    pre = _sc_gather_pre(pre_table, ids)

    w4 = jnp.concatenate([W_iouh0[:H], W_iouh1[:H],
                          W_fh0 + W_fh1, W_fh2 + W_fh3], axis=0)
    b4 = jnp.concatenate([b_iouh0[:H], b_iouh1[:H],
                          b_fh0 + b_fh1, b_fh2 + b_fh3]).reshape(1, 4 * H)
    w4_t = w4.T

    tids = jnp.transpose(tree_ids.reshape(NSTEPS, 3, BS), (0, 2, 1))
    tids = tids.astype(jnp.int32)

    h = jnp.zeros((BS, H), _F32)
    c = jnp.zeros((BS, H), _F32)
    for step in range(NSTEPS):
        tab_h, tab_c, src = _tc_step(h, c, pre, tids[step], w4_t, b4)
        h, c = _sc_gather_rows_pair(tab_h.reshape(2 * BS, H),
                                    tab_c.reshape(2 * BS, H),
                                    src.reshape(BS))
    return h.reshape(B, S, H)


# speedup vs baseline: 58.1944x; 1.3620x over previous
import jax
import jax.numpy as jnp
from jax import lax
from jax.experimental import pallas as pl
from jax.experimental.pallas import tpu as pltpu
from jax.experimental.pallas import tpu_sc as plsc

B, S, H, E, V, NSTEPS = 1024, 50, 64, 64, 100000, 10
BS = B * S
NW = 32
NPER = BS // NW
R = 200
NBLK = BS // R

_F32 = jnp.float32


def _sc_chunks():
    out, off = [], 0
    while off < NPER:
        sz = min(128, NPER - off)
        out.append((off, sz))
        off += sz
    return out


def _sc_gather_body(tab_hbm, idx_hbm, out_hbm, idx_v, bufs, sem):
    wid = lax.axis_index("s") * 2 + lax.axis_index("c")
    base = wid * NPER
    pltpu.sync_copy(idx_hbm.at[pl.ds(base, NPER)], idx_v)
    chunks = _sc_chunks()
    prev = None
    for t, (off, sz) in enumerate(chunks):
        cur = pltpu.async_copy(tab_hbm.at[idx_v.at[pl.ds(off, sz)]],
                               bufs.at[t % 2, pl.ds(0, sz)], sem)
        if prev is not None:
            poff, psz = chunks[t - 1]
            prev.wait()
            pltpu.sync_copy(bufs.at[(t - 1) % 2, pl.ds(0, psz)],
                            out_hbm.at[pl.ds(base + poff, psz)])
        prev = cur
    loff, lsz = chunks[-1]
    prev.wait()
    pltpu.sync_copy(bufs.at[(len(chunks) - 1) % 2, pl.ds(0, lsz)],
                    out_hbm.at[pl.ds(base + loff, lsz)])


def _sc_gather_rows(table, idx, width):
    mesh = plsc.VectorSubcoreMesh(core_axis_name="c", subcore_axis_name="s")
    return pl.kernel(
        _sc_gather_body,
        out_type=jax.ShapeDtypeStruct((BS, width), _F32),
        mesh=mesh,
        scratch_types=[
            pltpu.VMEM((NPER,), jnp.int32),
            pltpu.VMEM((2, 128, width), _F32),
            pltpu.SemaphoreType.DMA,
        ],
    )(table, idx)


def _pre_body(x_ref, wpre_ref, pre_ref):
    t = jax.lax.dot_general(x_ref[...], wpre_ref[...],
                            (((1,), (0,)), ((), ())),
                            preferred_element_type=_F32)
    col = lax.broadcasted_iota(jnp.int32, t.shape, 1)
    pre_ref[...] = jnp.where(
        col < 64, t,
        jnp.where(col < 128, jax.nn.sigmoid(t),
                  jnp.where(col < 192, jnp.tanh(t), t)))


def _tc_pre_table(emb, wpre_t):
    rp = 800
    return pl.pallas_call(
        _pre_body,
        grid=(V // rp,),
        in_specs=[pl.BlockSpec((rp, E), lambda k: (k, 0)),
                  pl.BlockSpec((E, 4 * H), lambda k: (0, 0))],
        out_specs=pl.BlockSpec((rp, 4 * H), lambda k: (k, 0)),
        out_shape=jax.ShapeDtypeStruct((V, 4 * H), _F32),
    )(emb, wpre_t)


def _step_body(hc_ref, pre_ref, tid_ref, w4_ref, b4_ref,
               tab_ref, src_ref, carry_ref):
    k = pl.program_id(0)
    hc_blk = hc_ref[...]
    h_blk = hc_blk[:, 0:H]
    c_blk = hc_blk[:, H:2 * H]
    pre = pre_ref[...]

    hh = jax.lax.dot_general(h_blk, w4_ref[...], (((1,), (0,)), ((), ())),
                             preferred_element_type=_F32) + b4_ref[...]
    hr, hl = hh[:, 0:H], hh[:, H:2 * H]
    fr, fl = hh[:, 2 * H:3 * H], hh[:, 3 * H:4 * H]

    ri = lax.broadcasted_iota(jnp.int32, (R, 1), 0)
    ebase = (ri // S) * S
    cj = lax.broadcasted_iota(jnp.int32, (R, R), 1)

    def onehot(idx_col):
        return (jnp.broadcast_to(ebase + idx_col, (R, R)) == cj).astype(_F32)

    idx_d = tid_ref[:, 0:1]
    o_d = onehot(idx_d)
    o_r = onehot(tid_ref[:, 1:2])
    o_l = onehot(tid_ref[:, 2:3])

    def gath(o, v):
        return jax.lax.dot_general(o, v, (((1,), (0,)), ((), ())),
                                   preferred_element_type=_F32)

    def scat(o, v):
        return jax.lax.dot_general(o, v, (((0,), (0,)), ((), ())),
                                   preferred_element_type=_F32)

    i_gate = jax.nn.sigmoid(pre[:, 0:H] + scat(o_r, hr) + scat(o_l, hl))
    o_gate = pre[:, H:2 * H]
    u_gate = pre[:, 2 * H:3 * H]
    fx_blk = pre[:, 3 * H:4 * H]
    f_gate = jax.nn.sigmoid(gath(o_d, fx_blk) + gath(o_r, fr) + gath(o_l, fl))
    c_new = i_gate * u_gate + scat(o_d, f_gate * c_blk)
    h_new = o_gate * jnp.tanh(c_new)

    tab_ref[0, :, 0:H] = h_new
    tab_ref[0, :, H:2 * H] = c_new
    tab_ref[1] = hc_blk

    mask = idx_d != 0
    mask_f = mask.astype(_F32)
    tri = (cj <= lax.broadcasted_iota(jnp.int32, (R, R), 0)).astype(_F32)
    cum = jax.lax.dot_general(tri, mask_f, (((1,), (0,)), ((), ())),
                              preferred_element_type=_F32).astype(jnp.int32)
    base = jnp.where(k == 0, 0, carry_ref[0])
    carry_ref[0] = base + jnp.sum(mask_f).astype(jnp.int32)
    rank = base + cum - 1
    src_ref[...] = jnp.where(mask, rank, BS + k * R + ri)


def _tc_step(hc, pre, tid, w4_t, b4):
    return pl.pallas_call(
        _step_body,
        grid=(NBLK,),
        in_specs=[
            pl.BlockSpec((R, 2 * H), lambda k: (k, 0)),
            pl.BlockSpec((R, 4 * H), lambda k: (k, 0)),
            pl.BlockSpec((R, 3), lambda k: (k, 0)),
            pl.BlockSpec((H, 4 * H), lambda k: (0, 0)),
            pl.BlockSpec((1, 4 * H), lambda k: (0, 0)),
        ],
        out_specs=[
            pl.BlockSpec((2, R, 2 * H), lambda k: (0, k, 0)),
            pl.BlockSpec((R, 1), lambda k: (k, 0)),
        ],
        out_shape=[
            jax.ShapeDtypeStruct((2, BS, 2 * H), _F32),
            jax.ShapeDtypeStruct((BS, 1), jnp.int32),
        ],
        scratch_shapes=[pltpu.SMEM((1,), jnp.int32)],
    )(hc, pre, tid, w4_t, b4)


def kernel(input_ids, tree_ids, emb, W_ioux, W_iouh0, b_iouh0, W_iouh1,
           b_iouh1, W_fx, W_fh0, b_fh0, W_fh1, b_fh1, W_fh2, b_fh2,
           W_fh3, b_fh3):
    ids = input_ids.reshape(BS).astype(jnp.int32)
    wpre_t = jnp.concatenate([W_ioux, W_fx], axis=0).T
    pre_table = _tc_pre_table(emb, wpre_t)
    pre = _sc_gather_rows(pre_table, ids, 4 * H)

    w4 = jnp.concatenate([W_iouh0[:H], W_iouh1[:H],
                          W_fh0 + W_fh1, W_fh2 + W_fh3], axis=0)
    b4 = jnp.concatenate([b_iouh0[:H], b_iouh1[:H],
                          b_fh0 + b_fh1, b_fh2 + b_fh3]).reshape(1, 4 * H)
    w4_t = w4.T

    tids = jnp.transpose(tree_ids.reshape(NSTEPS, 3, BS), (0, 2, 1))
    tids = tids.astype(jnp.int32)

    hc = jnp.zeros((BS, 2 * H), _F32)
    for step in range(NSTEPS):
        tab, src = _tc_step(hc, pre, tids[step], w4_t, b4)
        hc = _sc_gather_rows(tab.reshape(2 * BS, 2 * H), src.reshape(BS),
                             2 * H)
    return hc[:, 0:H].reshape(B, S, H)
